# baseline (speedup 1.0000x reference)
import functools
import math

import jax
import jax.numpy as jnp
from jax import lax
from jax.experimental import pallas as pl
from jax.experimental.pallas import tpu as pltpu
from jax.experimental.pallas import tpu_sc as plsc

_N0, _N1, _N2, _NNET = 50000, 12500, 3125, 20000
_SC_PARAMS = pltpu.CompilerParams(use_tc_tiling_on_sc=False,
                                  needs_layout_passes=False)


def _cdiv(a, b):
    return -(-a // b)


def _pick_seg(ept):
    best = None
    for seg in range(256, 8193, 128):
        nseg = _cdiv(ept, seg)
        key = (nseg * seg - ept, -seg)
        if best is None or key < best[0]:
            best = (key, seg, nseg)
    return best[1], best[2]


@functools.lru_cache(maxsize=None)
def _build_segsum(W, E, n_dst, R, with_counts):
    rn = _cdiv(_cdiv(n_dst, R), 128) * 128
    A = rn + 128
    NPAD = R * rn

    shared = A * W + (A * 16 if with_counts else 0)

    def _fits(seg, nb):
        cap = _cdiv(seg + nb * 128 + 16, 128) * 128
        per_tile = 2 * seg + 3 * cap + nb * 128 * W
        if with_counts:
            per_tile += 4096
        return 16 * per_tile + shared <= 1_980_000

    NB = 2
    while NB > 1 and not _fits(256, NB):
        NB -= 1
    smax = 256
    for seg in range(256, 8193, 128):
        if _fits(seg, NB):
            smax = seg
    ept0 = _cdiv(E, 16)
    best = None
    for seg in range(256, smax + 1, 128):
        nseg = _cdiv(ept0, seg)
        key = (nseg * seg - ept0, nseg)
        if best is None or key < best[0]:
            best = (key, seg, nseg)
    SEG, NSEGS = best[1], best[2]
    EPT = SEG * NSEGS
    EPAD = EPT * 16
    CAP = _cdiv(SEG + NB * 128 + 16, 128) * 128
    K = CAP // 128
    TRASH = CAP - 16
    RPASS = R // 2
    zshare = A // 16
    eshare = rn // 16

    mesh = plsc.VectorSubcoreMesh(core_axis_name="c", subcore_axis_name="s")
    out_type = [jax.ShapeDtypeStruct((NPAD, W), jnp.float32)]
    scratch = [
        pltpu.VMEM((SEG,), jnp.int32),
        pltpu.VMEM((SEG,), jnp.int32),
        pltpu.VMEM((CAP,), jnp.int32),
        pltpu.VMEM((CAP,), jnp.int32),
        pltpu.VMEM((K, 128), jnp.int32),
        pltpu.VMEM_SHARED((A, W), jnp.float32),
    ]
    scratch += [pltpu.VMEM((128, W), jnp.float32) for _ in range(NB)]
    scratch += [pltpu.SemaphoreType.DMA for _ in range(2 * NB)]
    if with_counts:
        out_type.append(jax.ShapeDtypeStruct((NPAD, 16), jnp.float32))
        scratch += [
            pltpu.VMEM((128, 16), jnp.float32),
            pltpu.VMEM((128, 16), jnp.float32),
            pltpu.VMEM_SHARED((A, 16), jnp.float32),
        ]
        scratch += [pltpu.SemaphoreType.DMA for _ in range(NB)]

    def body(table, srcp, dstp, *rest):
        nout = 2 if with_counts else 1
        if with_counts:
            sums, counts = rest[0], rest[1]
        else:
            sums = rest[0]
        sbuf, dbuf, csrc, cdst, cdst2, acc = rest[nout:nout + 6]
        rows = list(rest[nout + 6:nout + 6 + NB])
        gsem = list(rest[nout + 6 + NB:nout + 6 + 2 * NB])
        ssem = list(rest[nout + 6 + 2 * NB:nout + 6 + 3 * NB])
        if with_counts:
            cbuf, zc, cacc = rest[nout + 6 + 3 * NB:nout + 9 + 3 * NB]
            csem = list(rest[nout + 9 + 3 * NB:nout + 9 + 4 * NB])
        c = lax.axis_index("c")
        s = lax.axis_index("s")
        e0 = s * EPT
        iv = jnp.arange(16, dtype=jnp.int32)

        for q in range(RPASS):
            lo = (c * RPASS + q) * rn

            def zf(i, _):
                for k in range(W // 16):
                    rows[0][i, pl.ds(k * 16, 16)] = jnp.zeros((16,),
                                                              jnp.float32)
                if with_counts:
                    zc[i, :] = jnp.zeros((16,), jnp.float32)
                    cbuf[i, :] = jnp.ones((16,), jnp.float32)
                return 0
            lax.fori_loop(0, 128, zf, 0)
            z0 = pl.multiple_of(s * zshare, 8)
            nz, zrem = zshare // 128, zshare % 128
            for k in range(nz):
                pltpu.sync_copy(rows[0], acc.at[pl.ds(z0 + k * 128, 128)])
                if with_counts:
                    pltpu.sync_copy(zc, cacc.at[pl.ds(z0 + k * 128, 128)])
            if zrem:
                pltpu.sync_copy(rows[0].at[pl.ds(0, zrem)],
                                acc.at[pl.ds(z0 + nz * 128, zrem)])
                if with_counts:
                    pltpu.sync_copy(zc.at[pl.ds(0, zrem)],
                                    cacc.at[pl.ds(z0 + nz * 128, zrem)])
            plsc.subcore_barrier()

            def seg_body(g, _):
                base = pl.multiple_of(e0 + g * SEG, 128)
                pltpu.sync_copy(srcp.at[pl.ds(base, SEG)], sbuf)
                pltpu.sync_copy(dstp.at[pl.ds(base, SEG)], dbuf)

                def cmp(i, pos):
                    d = dbuf[pl.ds(i * 16, 16)]
                    sv = sbuf[pl.ds(i * 16, 16)]
                    m = (d >= lo) & (d < lo + rn)
                    ci = plsc.cumsum(m.astype(jnp.int32))
                    tgt = jnp.where(m, pos + ci - 1, TRASH + iv)
                    plsc.store_scatter(csrc, [tgt], sv)
                    plsc.store_scatter(cdst, [tgt], d - lo)
                    return pos + jnp.max(ci)
                pos = lax.fori_loop(0, SEG // 16, cmp, jnp.int32(0))

                for k in range(8 * NB):
                    csrc[pl.ds(pos + k * 16, 16)] = jnp.zeros((16,), jnp.int32)
                    cdst[pl.ds(pos + k * 16, 16)] = jnp.full((16,), rn,
                                                             jnp.int32)
                nch = (pos + 127) // 128
                nchg = (nch + NB - 1) // NB

                def rp(j, _):
                    for k in range(8):
                        cdst2[j, pl.ds(k * 16, 16)] = cdst[pl.ds(j * 128
                                                                 + k * 16, 16)]
                    return 0
                lax.fori_loop(0, nchg * NB, rp, 0)

                def grp(t, _):
                    j0 = t * NB
                    g0 = pltpu.async_copy(
                        table.at[csrc.at[pl.ds(j0 * 128, 128)]],
                        rows[0], gsem[0])
                    g0.wait()
                    g1 = pltpu.async_copy(
                        table.at[csrc.at[pl.ds((j0 + 1) * 128, 128)]],
                        rows[1], gsem[1])
                    d0 = pltpu.async_copy(rows[0], acc.at[cdst2.at[j0]],
                                          ssem[0], add=True)
                    if with_counts:
                        c0 = pltpu.async_copy(cbuf, cacc.at[cdst2.at[j0]],
                                              csem[0], add=True)
                        c0.wait()
                    d0.wait()
                    g1.wait()
                    d1 = pltpu.async_copy(rows[1], acc.at[cdst2.at[j0 + 1]],
                                          ssem[1], add=True)
                    if with_counts:
                        c1 = pltpu.async_copy(cbuf, cacc.at[cdst2.at[j0 + 1]],
                                              csem[1], add=True)
                        c1.wait()
                    d1.wait()
                    return 0
                lax.fori_loop(0, nchg, grp, 0)
                return 0
            lax.fori_loop(0, NSEGS, seg_body, 0)
            plsc.subcore_barrier()

            o0 = pl.multiple_of(s * eshare, 8)
            nef, erem = eshare // 128, eshare % 128
            for k in range(nef):
                pltpu.sync_copy(acc.at[pl.ds(o0 + k * 128, 128)], rows[0])
                pltpu.sync_copy(rows[0], sums.at[pl.ds(
                    pl.multiple_of(lo + o0 + k * 128, 8), 128)])
                if with_counts:
                    pltpu.sync_copy(cacc.at[pl.ds(o0 + k * 128, 128)], cbuf)
                    pltpu.sync_copy(cbuf, counts.at[pl.ds(
                        pl.multiple_of(lo + o0 + k * 128, 8), 128)])
            if erem:
                pltpu.sync_copy(acc.at[pl.ds(o0 + nef * 128, erem)],
                                rows[0].at[pl.ds(0, erem)])
                pltpu.sync_copy(rows[0].at[pl.ds(0, erem)], sums.at[pl.ds(
                    pl.multiple_of(lo + o0 + nef * 128, 8), erem)])
                if with_counts:
                    pltpu.sync_copy(cacc.at[pl.ds(o0 + nef * 128, erem)],
                                    cbuf.at[pl.ds(0, erem)])
                    pltpu.sync_copy(cbuf.at[pl.ds(0, erem)], counts.at[pl.ds(
                        pl.multiple_of(lo + o0 + nef * 128, 8), erem)])
            plsc.subcore_barrier()

    return pl.kernel(body, out_type=out_type, mesh=mesh,
                     compiler_params=_SC_PARAMS,
                     scratch_types=scratch), EPAD, NPAD


def _sc_gathersum(table, src, dst, n_dst, R, with_counts):
    E = src.shape[0]
    k, EPAD, NPAD = _build_segsum(table.shape[1], E, n_dst, R, with_counts)
    if EPAD > E:
        src = jnp.pad(src, (0, EPAD - E))
        dst = jnp.pad(dst, (0, EPAD - E), constant_values=NPAD)
    out = k(table, src, dst)
    if with_counts:
        return out[0][:n_dst], out[1][:n_dst, :1]
    return out[0][:n_dst]


@functools.lru_cache(maxsize=None)
def _build_segmax(W, E, n_dst):
    NT = 32
    rn = _cdiv(_cdiv(n_dst, NT), 16) * 16
    A = rn + 16
    NPAD = NT * rn
    SEG, NSEGS = _pick_seg(E)
    EPAD = SEG * NSEGS
    CAP = _cdiv(SEG + 2 * 128 + 16, 128) * 128
    TRASH = CAP - 16
    NEG = -3.0e38

    mesh = plsc.VectorSubcoreMesh(core_axis_name="c", subcore_axis_name="s")
    out_type = jax.ShapeDtypeStruct((NPAD, W), jnp.float32)
    scratch = [
        pltpu.VMEM((SEG,), jnp.int32),
        pltpu.VMEM((SEG,), jnp.int32),
        pltpu.VMEM((CAP,), jnp.int32),
        pltpu.VMEM((CAP,), jnp.int32),
        pltpu.VMEM((128, W), jnp.float32),
        pltpu.VMEM((128, W), jnp.float32),
        pltpu.VMEM((A, W), jnp.float32),
        pltpu.SemaphoreType.DMA,
        pltpu.SemaphoreType.DMA,
    ]

    def body(table, srcp, dstp, out, sbuf, dbuf, csrc, cdst, rows0, rows1,
             accm, sem0, sem1):
        rowsb = [rows0, rows1]
        semb = [sem0, sem1]
        c = lax.axis_index("c")
        s = lax.axis_index("s")
        wid = s * 2 + c
        lo = wid * rn
        iv = jnp.arange(16, dtype=jnp.int32)

        def init(i, _):
            for k in range(W // 16):
                accm[i, pl.ds(k * 16, 16)] = jnp.full((16,), NEG, jnp.float32)
            return 0
        lax.fori_loop(0, A, init, 0)

        def seg_body(g, _):
            base = pl.multiple_of(g * SEG, 128)
            pltpu.sync_copy(srcp.at[pl.ds(base, SEG)], sbuf)
            pltpu.sync_copy(dstp.at[pl.ds(base, SEG)], dbuf)

            def cmp(i, pos):
                d = dbuf[pl.ds(i * 16, 16)]
                sv = sbuf[pl.ds(i * 16, 16)]
                m = (d >= lo) & (d < lo + rn)
                ci = plsc.cumsum(m.astype(jnp.int32))
                tgt = jnp.where(m, pos + ci - 1, TRASH + iv)
                plsc.store_scatter(csrc, [tgt], sv)
                plsc.store_scatter(cdst, [tgt], d - lo)
                return pos + jnp.max(ci)
            pos = lax.fori_loop(0, SEG // 16, cmp, jnp.int32(0))

            for k in range(16):
                csrc[pl.ds(pos + k * 16, 16)] = jnp.zeros((16,), jnp.int32)
                cdst[pl.ds(pos + k * 16, 16)] = jnp.full((16,), rn, jnp.int32)
            nch = (pos + 127) // 128

            def gs(j, _):
                pltpu.async_copy(table.at[csrc.at[pl.ds(j * 128, 128)]],
                                 rowsb[0], semb[0]).wait()
                for gq in range(8):
                    dvec = cdst[pl.ds(j * 128 + gq * 16, 16)]
                    for l in range(16):
                        dl = jnp.max(jnp.where(iv == l, dvec, jnp.int32(-1)))
                        jj = gq * 16 + l
                        for k in range(W // 16):
                            a = accm[dl, pl.ds(k * 16, 16)]
                            r = rowsb[0][jj, pl.ds(k * 16, 16)]
                            accm[dl, pl.ds(k * 16, 16)] = jnp.maximum(a, r)
                return 0
            lax.fori_loop(0, nch, gs, 0)
            return 0
        lax.fori_loop(0, NSEGS, seg_body, 0)

        nef, erem = rn // 128, rn % 128
        for ch in range(nef + (1 if erem else 0)):
            cnt = 128 if ch < nef else erem

            def fin(i, _):
                for k in range(W // 16):
                    v = accm[ch * 128 + i, pl.ds(k * 16, 16)]
                    rows0[i, pl.ds(k * 16, 16)] = jnp.where(
                        v < -1.0e38, jnp.zeros((16,), jnp.float32), v)
                return 0
            lax.fori_loop(0, cnt, fin, 0)
            if cnt == 128:
                pltpu.sync_copy(rows0, out.at[pl.ds(
                    pl.multiple_of(lo + ch * 128, 8), 128)])
            else:
                pltpu.sync_copy(rows0.at[pl.ds(0, cnt)], out.at[pl.ds(
                    pl.multiple_of(lo + ch * 128, 8), cnt)])

    return pl.kernel(body, out_type=out_type, mesh=mesh,
                     compiler_params=_SC_PARAMS,
                     scratch_types=scratch), EPAD, NPAD


def _sc_segmax(table, src, dst, n_dst):
    E = src.shape[0]
    k, EPAD, NPAD = _build_segmax(table.shape[1], E, n_dst)
    if EPAD > E:
        src = jnp.pad(src, (0, EPAD - E))
        dst = jnp.pad(dst, (0, EPAD - E), constant_values=NPAD)
    return k(table, src, dst)[:n_dst]


def _sage_body(x_ref, s_ref, c_ref, ws_ref, wn_ref, b_ref, o_ref):
    m = s_ref[...] / jnp.maximum(c_ref[...], 1.0)
    a = jax.lax.dot_general(x_ref[...], ws_ref[...], (((1,), (1,)), ((), ())),
                            preferred_element_type=jnp.float32)
    a = a + jax.lax.dot_general(m, wn_ref[...], (((1,), (1,)), ((), ())),
                                preferred_element_type=jnp.float32)
    o_ref[...] = jnp.tanh(a + b_ref[...])


def _sage_tc(x, s, c, Ws, Wn, b, block=2048):
    n, din = x.shape
    dout = Ws.shape[0]
    grid = (n + block - 1) // block
    return pl.pallas_call(
        _sage_body,
        grid=(grid,),
        in_specs=[
            pl.BlockSpec((block, din), lambda i: (i, 0)),
            pl.BlockSpec((block, din), lambda i: (i, 0)),
            pl.BlockSpec((block, 1), lambda i: (i, 0)),
            pl.BlockSpec((dout, din), lambda i: (0, 0)),
            pl.BlockSpec((dout, din), lambda i: (0, 0)),
            pl.BlockSpec((1, dout), lambda i: (0, 0)),
        ],
        out_specs=pl.BlockSpec((block, dout), lambda i: (i, 0)),
        out_shape=jax.ShapeDtypeStruct((n, dout), jnp.float32),
    )(x, s, c, Ws, Wn, b.reshape(1, -1))


def _div_body(s_ref, c_ref, o_ref):
    o_ref[...] = s_ref[...] / jnp.maximum(c_ref[...], 1.0)


def _div_tc(s, c, block=4096):
    n, w = s.shape
    grid = (n + block - 1) // block
    return pl.pallas_call(
        _div_body,
        grid=(grid,),
        in_specs=[
            pl.BlockSpec((block, w), lambda i: (i, 0)),
            pl.BlockSpec((block, 1), lambda i: (i, 0)),
        ],
        out_specs=pl.BlockSpec((block, w), lambda i: (i, 0)),
        out_shape=jax.ShapeDtypeStruct((n, w), jnp.float32),
    )(s, c)


def _mlp_body(xx_ref, w1_ref, b1_ref, w2_ref, o_ref):
    h = jax.lax.dot_general(xx_ref[...], w1_ref[...], (((1,), (1,)), ((), ())),
                            preferred_element_type=jnp.float32)
    h = jnp.tanh(h + b1_ref[...])
    o_ref[...] = jax.lax.dot_general(h, w2_ref[...], (((1,), (1,)), ((), ())),
                                     preferred_element_type=jnp.float32)


def _mlp_tc(xx, W1, b1, W2, b2, block=2048):
    n, din = xx.shape
    h = W1.shape[0]
    grid = (n + block - 1) // block
    out = pl.pallas_call(
        _mlp_body,
        grid=(grid,),
        in_specs=[
            pl.BlockSpec((block, din), lambda i: (i, 0)),
            pl.BlockSpec((h, din), lambda i: (0, 0)),
            pl.BlockSpec((1, h), lambda i: (0, 0)),
            pl.BlockSpec((1, h), lambda i: (0, 0)),
        ],
        out_specs=pl.BlockSpec((block, 1), lambda i: (i, 0)),
        out_shape=jax.ShapeDtypeStruct((n, 1), jnp.float32),
    )(xx, W1, b1.reshape(1, -1), W2)
    return out + b2


def kernel(x0, x_net, to0, to1, to2, down01_src, down01_dst, down12_src,
           down12_dst, up21_src, up21_dst, up10_src, up10_dst, conn_src,
           conn_dst, W_self_0, W_neigh_0, b_0, W_self_1, W_neigh_1, b_1,
           W_self_2, W_neigh_2, b_2, W_self_3, W_neigh_3, b_3, W_self_4,
           W_neigh_4, b_4, mlp_W1, mlp_b1, mlp_W2, mlp_b2):
    s0, c0 = _sc_gathersum(x0, to0[0], to0[1], _N0, 4, True)
    x0_ = _sage_tc(x0, s0, c0, W_self_0, W_neigh_0, b_0)
    sx1, c01 = _sc_gathersum(x0_, down01_src, down01_dst, _N1, 2, True)
    x1 = _div_tc(sx1, c01)
    s1, c1 = _sc_gathersum(x1, to1[0], to1[1], _N1, 2, True)
    x1_ = _sage_tc(x1, s1, c1, W_self_1, W_neigh_1, b_1)
    sx2, c12 = _sc_gathersum(x1_, down12_src, down12_dst, _N2, 2, True)
    x2 = _div_tc(sx2, c12)
    s2, c2 = _sc_gathersum(x2, to2[0], to2[1], _N2, 2, True)
    x2_ = _sage_tc(x2, s2, c2, W_self_2, W_neigh_2, b_2)

    l1 = _sc_gathersum(x2_, up21_src, up21_dst, _N1, 2, False)
    r1 = _sc_gathersum(x1_, up21_dst, up21_dst, _N1, 2, False)
    x1__ = jnp.concatenate([l1, r1], axis=1)
    s3 = _sc_gathersum(x1__, to1[0], to1[1], _N1, 2, False)
    x1u = _sage_tc(x1__, s3, c1, W_self_3, W_neigh_3, b_3)
    l0 = _sc_gathersum(x1u, up10_src, up10_dst, _N0, 4, False)
    r0 = _sc_gathersum(x0_, up10_dst, up10_dst, _N0, 4, False)
    x0__ = jnp.concatenate([l0, r0], axis=1)
    s4 = _sc_gathersum(x0__, to0[0], to0[1], _N0, 8, False)
    x0u = _sage_tc(x0__, s4, c0, W_self_4, W_neigh_4, b_4)

    y = _sc_segmax(x0u, conn_src, conn_dst, _NNET)
    xx = jnp.concatenate([y, x_net], axis=1)
    return _mlp_tc(xx, mlp_W1, mlp_b1, mlp_W2, mlp_b2)


# speedup vs baseline: 2.1093x; 2.1093x over previous
import functools
import math

import jax
import jax.numpy as jnp
from jax import lax
from jax.experimental import pallas as pl
from jax.experimental.pallas import tpu as pltpu
from jax.experimental.pallas import tpu_sc as plsc

_N0, _N1, _N2, _NNET = 50000, 12500, 3125, 20000
_SC_PARAMS = pltpu.CompilerParams(use_tc_tiling_on_sc=False,
                                  needs_layout_passes=False)


def _cdiv(a, b):
    return -(-a // b)


def _pick_seg(ept):
    best = None
    for seg in range(256, 8193, 128):
        nseg = _cdiv(ept, seg)
        key = (nseg * seg - ept, -seg)
        if best is None or key < best[0]:
            best = (key, seg, nseg)
    return best[1], best[2]


@functools.lru_cache(maxsize=None)
def _build_segsum(W, E, n_dst, R, with_counts):
    rn = _cdiv(_cdiv(n_dst, R), 128) * 128
    A = rn + 128
    NPAD = R * rn

    shared = A * W + (A * 16 if with_counts else 0)

    def _fits(seg, nb):
        cap = _cdiv(seg + nb * 128 + 16, 128) * 128
        per_tile = 2 * seg + 3 * cap + nb * 128 * W
        if with_counts:
            per_tile += 4096
        return 16 * per_tile + shared <= 1_980_000

    NB = 1
    while NB > 1 and not _fits(256, NB):
        NB -= 1
    smax = 256
    for seg in range(256, 8193, 128):
        if _fits(seg, NB):
            smax = seg
    ept0 = _cdiv(E, 16)
    best = None
    for seg in range(256, smax + 1, 128):
        nseg = _cdiv(ept0, seg)
        key = (nseg * seg - ept0, nseg)
        if best is None or key < best[0]:
            best = (key, seg, nseg)
    SEG, NSEGS = best[1], best[2]
    EPT = SEG * NSEGS
    EPAD = EPT * 16
    CAP = _cdiv(SEG + NB * 128 + 16, 128) * 128
    K = CAP // 128
    TRASH = CAP - 16
    RPASS = R // 2
    zshare = A // 16
    eshare = rn // 16

    mesh = plsc.VectorSubcoreMesh(core_axis_name="c", subcore_axis_name="s")
    out_type = [jax.ShapeDtypeStruct((NPAD, W), jnp.float32)]
    scratch = [
        pltpu.VMEM((SEG,), jnp.int32),
        pltpu.VMEM((SEG,), jnp.int32),
        pltpu.VMEM((CAP,), jnp.int32),
        pltpu.VMEM((CAP,), jnp.int32),
        pltpu.VMEM((K, 128), jnp.int32),
        pltpu.VMEM_SHARED((A, W), jnp.float32),
    ]
    scratch += [pltpu.VMEM((128, W), jnp.float32) for _ in range(NB)]
    scratch += [pltpu.SemaphoreType.DMA for _ in range(2 * NB)]
    if with_counts:
        out_type.append(jax.ShapeDtypeStruct((NPAD, 16), jnp.float32))
        scratch += [
            pltpu.VMEM((128, 16), jnp.float32),
            pltpu.VMEM((128, 16), jnp.float32),
            pltpu.VMEM_SHARED((A, 16), jnp.float32),
        ]
        scratch += [pltpu.SemaphoreType.DMA for _ in range(NB)]

    def body(table, srcp, dstp, *rest):
        nout = 2 if with_counts else 1
        if with_counts:
            sums, counts = rest[0], rest[1]
        else:
            sums = rest[0]
        sbuf, dbuf, csrc, cdst, cdst2, acc = rest[nout:nout + 6]
        rows = list(rest[nout + 6:nout + 6 + NB])
        gsem = list(rest[nout + 6 + NB:nout + 6 + 2 * NB])
        ssem = list(rest[nout + 6 + 2 * NB:nout + 6 + 3 * NB])
        if with_counts:
            cbuf, zc, cacc = rest[nout + 6 + 3 * NB:nout + 9 + 3 * NB]
            csem = list(rest[nout + 9 + 3 * NB:nout + 9 + 4 * NB])
        c = lax.axis_index("c")
        s = lax.axis_index("s")
        e0 = s * EPT
        iv = jnp.arange(16, dtype=jnp.int32)

        for q in range(RPASS):
            lo = (c * RPASS + q) * rn

            def zf(i, _):
                for k in range(W // 16):
                    rows[0][i, pl.ds(k * 16, 16)] = jnp.zeros((16,),
                                                              jnp.float32)
                if with_counts:
                    zc[i, :] = jnp.zeros((16,), jnp.float32)
                    cbuf[i, :] = jnp.ones((16,), jnp.float32)
                return 0
            lax.fori_loop(0, 128, zf, 0)
            z0 = pl.multiple_of(s * zshare, 8)
            nz, zrem = zshare // 128, zshare % 128
            for k in range(nz):
                pltpu.sync_copy(rows[0], acc.at[pl.ds(z0 + k * 128, 128)])
                if with_counts:
                    pltpu.sync_copy(zc, cacc.at[pl.ds(z0 + k * 128, 128)])
            if zrem:
                pltpu.sync_copy(rows[0].at[pl.ds(0, zrem)],
                                acc.at[pl.ds(z0 + nz * 128, zrem)])
                if with_counts:
                    pltpu.sync_copy(zc.at[pl.ds(0, zrem)],
                                    cacc.at[pl.ds(z0 + nz * 128, zrem)])
            plsc.subcore_barrier()

            def seg_body(g, _):
                base = pl.multiple_of(e0 + g * SEG, 128)
                pltpu.sync_copy(srcp.at[pl.ds(base, SEG)], sbuf)
                pltpu.sync_copy(dstp.at[pl.ds(base, SEG)], dbuf)

                def cmp(i, pos):
                    d = dbuf[pl.ds(i * 16, 16)]
                    sv = sbuf[pl.ds(i * 16, 16)]
                    m = (d >= lo) & (d < lo + rn)
                    ci = plsc.cumsum(m.astype(jnp.int32))
                    tgt = jnp.where(m, pos + ci - 1, TRASH + iv)
                    plsc.store_scatter(csrc, [tgt], sv)
                    plsc.store_scatter(cdst, [tgt], d - lo)
                    return pos + jnp.max(ci)
                pos = lax.fori_loop(0, SEG // 16, cmp, jnp.int32(0))

                for k in range(8 * NB):
                    csrc[pl.ds(pos + k * 16, 16)] = jnp.zeros((16,), jnp.int32)
                    cdst[pl.ds(pos + k * 16, 16)] = jnp.full((16,), rn,
                                                             jnp.int32)
                nch = (pos + 127) // 128
                nchg = (nch + NB - 1) // NB

                def rp(j, _):
                    for k in range(8):
                        cdst2[j, pl.ds(k * 16, 16)] = cdst[pl.ds(j * 128
                                                                 + k * 16, 16)]
                    return 0
                lax.fori_loop(0, nchg * NB, rp, 0)

                def grp(j, _):
                    pltpu.async_copy(
                        table.at[csrc.at[pl.ds(j * 128, 128)]],
                        rows[0], gsem[0]).wait()
                    pltpu.sync_copy(rows[0], acc.at[cdst2.at[j]], add=True)
                    if with_counts:
                        pltpu.sync_copy(cbuf, cacc.at[cdst2.at[j]], add=True)
                    return 0
                lax.fori_loop(0, nch, grp, 0)
                return 0
            lax.fori_loop(0, NSEGS, seg_body, 0)
            plsc.subcore_barrier()

            o0 = pl.multiple_of(s * eshare, 8)
            nef, erem = eshare // 128, eshare % 128
            for k in range(nef):
                pltpu.sync_copy(acc.at[pl.ds(o0 + k * 128, 128)], rows[0])
                pltpu.sync_copy(rows[0], sums.at[pl.ds(
                    pl.multiple_of(lo + o0 + k * 128, 8), 128)])
                if with_counts:
                    pltpu.sync_copy(cacc.at[pl.ds(o0 + k * 128, 128)], cbuf)
                    pltpu.sync_copy(cbuf, counts.at[pl.ds(
                        pl.multiple_of(lo + o0 + k * 128, 8), 128)])
            if erem:
                pltpu.sync_copy(acc.at[pl.ds(o0 + nef * 128, erem)],
                                rows[0].at[pl.ds(0, erem)])
                pltpu.sync_copy(rows[0].at[pl.ds(0, erem)], sums.at[pl.ds(
                    pl.multiple_of(lo + o0 + nef * 128, 8), erem)])
                if with_counts:
                    pltpu.sync_copy(cacc.at[pl.ds(o0 + nef * 128, erem)],
                                    cbuf.at[pl.ds(0, erem)])
                    pltpu.sync_copy(cbuf.at[pl.ds(0, erem)], counts.at[pl.ds(
                        pl.multiple_of(lo + o0 + nef * 128, 8), erem)])
            plsc.subcore_barrier()

    return pl.kernel(body, out_type=out_type, mesh=mesh,
                     compiler_params=_SC_PARAMS,
                     scratch_types=scratch), EPAD, NPAD


def _sc_gathersum(table, src, dst, n_dst, R, with_counts):
    E = src.shape[0]
    k, EPAD, NPAD = _build_segsum(table.shape[1], E, n_dst, R, with_counts)
    if EPAD > E:
        src = jnp.pad(src, (0, EPAD - E))
        dst = jnp.pad(dst, (0, EPAD - E), constant_values=NPAD)
    out = k(table, src, dst)
    if with_counts:
        return out[0][:n_dst], out[1][:n_dst, :1]
    return out[0][:n_dst]


@functools.lru_cache(maxsize=None)
def _build_segmax(W, E, n_dst):
    NT = 32
    rn = _cdiv(_cdiv(n_dst, NT), 16) * 16
    A = rn + 16
    NPAD = NT * rn
    SEG, NSEGS = _pick_seg(E)
    EPAD = SEG * NSEGS
    CAP = _cdiv(SEG + 2 * 128 + 16, 128) * 128
    TRASH = CAP - 16
    NEG = -3.0e38

    mesh = plsc.VectorSubcoreMesh(core_axis_name="c", subcore_axis_name="s")
    out_type = jax.ShapeDtypeStruct((NPAD, W), jnp.float32)
    scratch = [
        pltpu.VMEM((SEG,), jnp.int32),
        pltpu.VMEM((SEG,), jnp.int32),
        pltpu.VMEM((CAP,), jnp.int32),
        pltpu.VMEM((CAP,), jnp.int32),
        pltpu.VMEM((128, W), jnp.float32),
        pltpu.VMEM((128, W), jnp.float32),
        pltpu.VMEM((A, W), jnp.float32),
        pltpu.SemaphoreType.DMA,
        pltpu.SemaphoreType.DMA,
    ]

    def body(table, srcp, dstp, out, sbuf, dbuf, csrc, cdst, rows0, rows1,
             accm, sem0, sem1):
        rowsb = [rows0, rows1]
        semb = [sem0, sem1]
        c = lax.axis_index("c")
        s = lax.axis_index("s")
        wid = s * 2 + c
        lo = wid * rn
        iv = jnp.arange(16, dtype=jnp.int32)

        def init(i, _):
            for k in range(W // 16):
                accm[i, pl.ds(k * 16, 16)] = jnp.full((16,), NEG, jnp.float32)
            return 0
        lax.fori_loop(0, A, init, 0)

        def seg_body(g, _):
            base = pl.multiple_of(g * SEG, 128)
            pltpu.sync_copy(srcp.at[pl.ds(base, SEG)], sbuf)
            pltpu.sync_copy(dstp.at[pl.ds(base, SEG)], dbuf)

            def cmp(i, pos):
                d = dbuf[pl.ds(i * 16, 16)]
                sv = sbuf[pl.ds(i * 16, 16)]
                m = (d >= lo) & (d < lo + rn)
                ci = plsc.cumsum(m.astype(jnp.int32))
                tgt = jnp.where(m, pos + ci - 1, TRASH + iv)
                plsc.store_scatter(csrc, [tgt], sv)
                plsc.store_scatter(cdst, [tgt], d - lo)
                return pos + jnp.max(ci)
            pos = lax.fori_loop(0, SEG // 16, cmp, jnp.int32(0))

            for k in range(16):
                csrc[pl.ds(pos + k * 16, 16)] = jnp.zeros((16,), jnp.int32)
                cdst[pl.ds(pos + k * 16, 16)] = jnp.full((16,), rn, jnp.int32)
            nch = (pos + 127) // 128

            def gs(j, _):
                pltpu.async_copy(table.at[csrc.at[pl.ds(j * 128, 128)]],
                                 rowsb[0], semb[0]).wait()
                for gq in range(8):
                    dvec = cdst[pl.ds(j * 128 + gq * 16, 16)]
                    for l in range(16):
                        dl = jnp.max(jnp.where(iv == l, dvec, jnp.int32(-1)))
                        jj = gq * 16 + l
                        for k in range(W // 16):
                            a = accm[dl, pl.ds(k * 16, 16)]
                            r = rowsb[0][jj, pl.ds(k * 16, 16)]
                            accm[dl, pl.ds(k * 16, 16)] = jnp.maximum(a, r)
                return 0
            lax.fori_loop(0, nch, gs, 0)
            return 0
        lax.fori_loop(0, NSEGS, seg_body, 0)

        nef, erem = rn // 128, rn % 128
        for ch in range(nef + (1 if erem else 0)):
            cnt = 128 if ch < nef else erem

            def fin(i, _):
                for k in range(W // 16):
                    v = accm[ch * 128 + i, pl.ds(k * 16, 16)]
                    rows0[i, pl.ds(k * 16, 16)] = jnp.where(
                        v < -1.0e38, jnp.zeros((16,), jnp.float32), v)
                return 0
            lax.fori_loop(0, cnt, fin, 0)
            if cnt == 128:
                pltpu.sync_copy(rows0, out.at[pl.ds(
                    pl.multiple_of(lo + ch * 128, 8), 128)])
            else:
                pltpu.sync_copy(rows0.at[pl.ds(0, cnt)], out.at[pl.ds(
                    pl.multiple_of(lo + ch * 128, 8), cnt)])

    return pl.kernel(body, out_type=out_type, mesh=mesh,
                     compiler_params=_SC_PARAMS,
                     scratch_types=scratch), EPAD, NPAD


def _sc_segmax(table, src, dst, n_dst):
    E = src.shape[0]
    k, EPAD, NPAD = _build_segmax(table.shape[1], E, n_dst)
    if EPAD > E:
        src = jnp.pad(src, (0, EPAD - E))
        dst = jnp.pad(dst, (0, EPAD - E), constant_values=NPAD)
    return k(table, src, dst)[:n_dst]


def _sage_body(x_ref, s_ref, c_ref, ws_ref, wn_ref, b_ref, o_ref):
    m = s_ref[...] / jnp.maximum(c_ref[...], 1.0)
    a = jax.lax.dot_general(x_ref[...], ws_ref[...], (((1,), (1,)), ((), ())),
                            preferred_element_type=jnp.float32)
    a = a + jax.lax.dot_general(m, wn_ref[...], (((1,), (1,)), ((), ())),
                                preferred_element_type=jnp.float32)
    o_ref[...] = jnp.tanh(a + b_ref[...])


def _sage_tc(x, s, c, Ws, Wn, b, block=2048):
    n, din = x.shape
    dout = Ws.shape[0]
    grid = (n + block - 1) // block
    return pl.pallas_call(
        _sage_body,
        grid=(grid,),
        in_specs=[
            pl.BlockSpec((block, din), lambda i: (i, 0)),
            pl.BlockSpec((block, din), lambda i: (i, 0)),
            pl.BlockSpec((block, 1), lambda i: (i, 0)),
            pl.BlockSpec((dout, din), lambda i: (0, 0)),
            pl.BlockSpec((dout, din), lambda i: (0, 0)),
            pl.BlockSpec((1, dout), lambda i: (0, 0)),
        ],
        out_specs=pl.BlockSpec((block, dout), lambda i: (i, 0)),
        out_shape=jax.ShapeDtypeStruct((n, dout), jnp.float32),
    )(x, s, c, Ws, Wn, b.reshape(1, -1))


def _div_body(s_ref, c_ref, o_ref):
    o_ref[...] = s_ref[...] / jnp.maximum(c_ref[...], 1.0)


def _div_tc(s, c, block=4096):
    n, w = s.shape
    grid = (n + block - 1) // block
    return pl.pallas_call(
        _div_body,
        grid=(grid,),
        in_specs=[
            pl.BlockSpec((block, w), lambda i: (i, 0)),
            pl.BlockSpec((block, 1), lambda i: (i, 0)),
        ],
        out_specs=pl.BlockSpec((block, w), lambda i: (i, 0)),
        out_shape=jax.ShapeDtypeStruct((n, w), jnp.float32),
    )(s, c)


def _mlp_body(xx_ref, w1_ref, b1_ref, w2_ref, o_ref):
    h = jax.lax.dot_general(xx_ref[...], w1_ref[...], (((1,), (1,)), ((), ())),
                            preferred_element_type=jnp.float32)
    h = jnp.tanh(h + b1_ref[...])
    o_ref[...] = jax.lax.dot_general(h, w2_ref[...], (((1,), (1,)), ((), ())),
                                     preferred_element_type=jnp.float32)


def _mlp_tc(xx, W1, b1, W2, b2, block=2048):
    n, din = xx.shape
    h = W1.shape[0]
    grid = (n + block - 1) // block
    out = pl.pallas_call(
        _mlp_body,
        grid=(grid,),
        in_specs=[
            pl.BlockSpec((block, din), lambda i: (i, 0)),
            pl.BlockSpec((h, din), lambda i: (0, 0)),
            pl.BlockSpec((1, h), lambda i: (0, 0)),
            pl.BlockSpec((1, h), lambda i: (0, 0)),
        ],
        out_specs=pl.BlockSpec((block, 1), lambda i: (i, 0)),
        out_shape=jax.ShapeDtypeStruct((n, 1), jnp.float32),
    )(xx, W1, b1.reshape(1, -1), W2)
    return out + b2


def kernel(x0, x_net, to0, to1, to2, down01_src, down01_dst, down12_src,
           down12_dst, up21_src, up21_dst, up10_src, up10_dst, conn_src,
           conn_dst, W_self_0, W_neigh_0, b_0, W_self_1, W_neigh_1, b_1,
           W_self_2, W_neigh_2, b_2, W_self_3, W_neigh_3, b_3, W_self_4,
           W_neigh_4, b_4, mlp_W1, mlp_b1, mlp_W2, mlp_b2):
    s0, c0 = _sc_gathersum(x0, to0[0], to0[1], _N0, 4, True)
    x0_ = _sage_tc(x0, s0, c0, W_self_0, W_neigh_0, b_0)
    sx1, c01 = _sc_gathersum(x0_, down01_src, down01_dst, _N1, 2, True)
    x1 = _div_tc(sx1, c01)
    s1, c1 = _sc_gathersum(x1, to1[0], to1[1], _N1, 2, True)
    x1_ = _sage_tc(x1, s1, c1, W_self_1, W_neigh_1, b_1)
    sx2, c12 = _sc_gathersum(x1_, down12_src, down12_dst, _N2, 2, True)
    x2 = _div_tc(sx2, c12)
    s2, c2 = _sc_gathersum(x2, to2[0], to2[1], _N2, 2, True)
    x2_ = _sage_tc(x2, s2, c2, W_self_2, W_neigh_2, b_2)

    l1 = _sc_gathersum(x2_, up21_src, up21_dst, _N1, 2, False)
    r1 = _sc_gathersum(x1_, up21_dst, up21_dst, _N1, 2, False)
    x1__ = jnp.concatenate([l1, r1], axis=1)
    s3 = _sc_gathersum(x1__, to1[0], to1[1], _N1, 2, False)
    x1u = _sage_tc(x1__, s3, c1, W_self_3, W_neigh_3, b_3)
    l0 = _sc_gathersum(x1u, up10_src, up10_dst, _N0, 4, False)
    r0 = _sc_gathersum(x0_, up10_dst, up10_dst, _N0, 4, False)
    x0__ = jnp.concatenate([l0, r0], axis=1)
    s4 = _sc_gathersum(x0__, to0[0], to0[1], _N0, 8, False)
    x0u = _sage_tc(x0__, s4, c0, W_self_4, W_neigh_4, b_4)

    y = _sc_segmax(x0u, conn_src, conn_dst, _NNET)
    xx = jnp.concatenate([y, x_net], axis=1)
    return _mlp_tc(xx, mlp_W1, mlp_b1, mlp_W2, mlp_b2)
